# baseline (speedup 1.0000x reference)
import jax
import jax.numpy as jnp
from jax.experimental import pallas as pl
from jax.experimental.pallas import tpu as pltpu


def _copy_body(tab_ref, out_ref):
    out_ref[...] = tab_ref[...]


def kernel(x, pos_table):
    b, n = x.shape[0], x.shape[1]
    d = pos_table.shape[1]
    bn = 1024
    nb = n // bn
    out = pl.pallas_call(
        _copy_body,
        grid=(nb, b),
        in_specs=[pl.BlockSpec((bn, d), lambda i, j: (i, 0))],
        out_specs=pl.BlockSpec((bn, d), lambda i, j: (j * nb + i, 0)),
        out_shape=jax.ShapeDtypeStruct((b * n, d), pos_table.dtype),
        compiler_params=pltpu.CompilerParams(
            dimension_semantics=("arbitrary", "arbitrary")
        ),
    )(pos_table)
    return out.reshape(b, n, d)
